# baseline (speedup 1.0000x reference)
import functools

import jax
import jax.numpy as jnp
from jax import lax
from jax.experimental import pallas as pl
from jax.experimental.pallas import tpu as pltpu
from jax.experimental.pallas import tpu_sc as plsc

N = 10000
IN_CH = 128
HID_CH = 256
OUT_CH = 128

NPAD = 10240
E_RAW = 320000 + N
CHUNK = 256
N_TILES = 16
EPAD = 335872
PER_TILE = EPAD // N_TILES
CHUNKS_PER_TILE = PER_TILE // CHUNK
ROWS_PER_TILE = NPAD // N_TILES
RB = 1024
N_RB = NPAD // RB


def _sc_agg_body(edge_split, skip_gather, table, idt, zeros, out, acc,
                 idt0, idt1, idt2, idt3, rows, gsem, ssem,
                 lsem0, lsem1, lsem2, lsem3):
    c = lax.axis_index("c")
    s = lax.axis_index("s")
    idts = [idt0, idt1, idt2, idt3]
    lsems = [lsem0, lsem1, lsem2, lsem3]

    r0 = s * ROWS_PER_TILE
    pltpu.sync_copy(zeros.at[pl.ds(r0, ROWS_PER_TILE)],
                    acc.at[pl.ds(r0, ROWS_PER_TILE)])

    if edge_split:
        n_chunks = CHUNKS_PER_TILE // 2
        cchunk = c * (EPAD // 2 // CHUNK)
    else:
        n_chunks = CHUNKS_PER_TILE
        cchunk = c * (EPAD // CHUNK)

    if skip_gather:
        pltpu.async_copy(table.at[pl.ds(0, CHUNK)], rows, gsem).wait()

    def idt_start(j, q):
        off = (cchunk + s * n_chunks + j) * (2 * CHUNK)
        pltpu.async_copy(idt.at[pl.ds(off, 2 * CHUNK)], idts[q], lsems[q])

    def idt_wait(q):
        pltpu.make_async_copy(idt.at[pl.ds(0, 2 * CHUNK)], idts[q],
                              lsems[q]).wait()

    def scatter_wait():
        pltpu.make_async_copy(rows, acc.at[idt0.at[pl.ds(CHUNK, CHUNK)]],
                              ssem).wait()

    idt_start(0, 0)
    idt_start(1, 1)
    idt_start(2, 2)
    plsc.subcore_barrier()

    def block(j, q, first=False):
        if not first:
            scatter_wait()
        idt_start(jnp.minimum(j + 3, n_chunks - 1), (q + 3) % 4)
        idt_wait(q)
        if not skip_gather:
            pltpu.async_copy(table.at[idts[q].at[pl.ds(0, CHUNK)]], rows,
                             gsem).wait()
        pltpu.async_copy(rows, acc.at[idts[q].at[pl.ds(CHUNK, CHUNK)]],
                         ssem, add=True)

    block(0, 0, first=True)
    block(1, 1)
    block(2, 2)
    block(3, 3)

    def group(p, carry):
        base = 4 * p
        block(base, 0)
        block(base + 1, 1)
        block(base + 2, 2)
        block(base + 3, 3)
        return carry

    lax.fori_loop(1, n_chunks // 4, group, 0)
    for j in range(n_chunks - (n_chunks % 4), n_chunks):
        block(j, j % 4)
    scatter_wait()
    for q in (n_chunks % 4, (n_chunks + 1) % 4, (n_chunks + 2) % 4):
        idt_wait(q)
    plsc.subcore_barrier()

    pltpu.sync_copy(acc.at[pl.ds(r0, ROWS_PER_TILE)],
                    out.at[pl.ds(c * NPAD + r0, ROWS_PER_TILE)])


@functools.partial(jax.jit, static_argnums=(0, 1, 2))
def _sc_agg(h, edge_split, skip_gather, table, idt):
    zeros = jnp.zeros((NPAD, h), jnp.float32)
    mesh = plsc.VectorSubcoreMesh(core_axis_name="c", subcore_axis_name="s")
    return pl.kernel(
        functools.partial(_sc_agg_body, edge_split, skip_gather),
        out_type=jax.ShapeDtypeStruct((2 * NPAD, h), jnp.float32),
        mesh=mesh,
        scratch_types=[
            pltpu.VMEM_SHARED((NPAD, h), jnp.float32),
            pltpu.VMEM((2 * CHUNK,), jnp.int32),
            pltpu.VMEM((2 * CHUNK,), jnp.int32),
            pltpu.VMEM((2 * CHUNK,), jnp.int32),
            pltpu.VMEM((2 * CHUNK,), jnp.int32),
            pltpu.VMEM((CHUNK, h), jnp.float32),
            pltpu.SemaphoreType.DMA,
            pltpu.SemaphoreType.DMA,
            pltpu.SemaphoreType.DMA,
            pltpu.SemaphoreType.DMA,
            pltpu.SemaphoreType.DMA,
            pltpu.SemaphoreType.DMA,
        ],
    )(table, idt, zeros)


def _tc_first_body(x_ref, deg_ref, w_ref, o_ref):
    dinv = lax.rsqrt(jnp.maximum(deg_ref[...], 1.0))
    t = x_ref[...] * dinv
    o_ref[...] = jnp.dot(t, w_ref[...], preferred_element_type=jnp.float32)


def _tc_first(x_pad, deg, w1):
    return pl.pallas_call(
        _tc_first_body,
        grid=(2, N_RB),
        in_specs=[
            pl.BlockSpec((RB, IN_CH), lambda c, i: (i, 0)),
            pl.BlockSpec((RB, 1), lambda c, i: (i, 0)),
            pl.BlockSpec((IN_CH, HID_CH // 2), lambda c, i: (0, c)),
        ],
        out_specs=pl.BlockSpec((RB, HID_CH // 2), lambda c, i: (c * N_RB + i, 0)),
        out_shape=jax.ShapeDtypeStruct((2 * NPAD, HID_CH // 2), jnp.float32),
    )(x_pad, deg, w1)


def _tc_mid_body(lo_ref, hi_ref, deg_ref, b_ref, w_ref, o_ref):
    dinv = lax.rsqrt(jnp.maximum(deg_ref[...], 1.0))
    s = jnp.concatenate([lo_ref[...], hi_ref[...]], axis=1)
    u = jnp.maximum(s * dinv + b_ref[...], 0.0)
    t = u * dinv
    o_ref[...] = jnp.dot(t, w_ref[...], preferred_element_type=jnp.float32)


def _tc_mid(s_cat, deg, b, w):
    h = HID_CH // 2
    return pl.pallas_call(
        _tc_mid_body,
        grid=(2, N_RB),
        in_specs=[
            pl.BlockSpec((RB, h), lambda c, i: (i, 0)),
            pl.BlockSpec((RB, h), lambda c, i: (N_RB + i, 0)),
            pl.BlockSpec((RB, 1), lambda c, i: (i, 0)),
            pl.BlockSpec((1, HID_CH), lambda c, i: (0, 0)),
            pl.BlockSpec((HID_CH, h), lambda c, i: (0, c)),
        ],
        out_specs=pl.BlockSpec((RB, h), lambda c, i: (c * N_RB + i, 0)),
        out_shape=jax.ShapeDtypeStruct((2 * NPAD, h), jnp.float32),
    )(s_cat, s_cat, deg, b.reshape(1, -1), w)


def _tc_mid_full(s_cat, deg, b, w):
    h = HID_CH // 2
    return pl.pallas_call(
        _tc_mid_body,
        grid=(N_RB,),
        in_specs=[
            pl.BlockSpec((RB, h), lambda i: (i, 0)),
            pl.BlockSpec((RB, h), lambda i: (N_RB + i, 0)),
            pl.BlockSpec((RB, 1), lambda i: (i, 0)),
            pl.BlockSpec((1, HID_CH), lambda i: (0, 0)),
            pl.BlockSpec((HID_CH, OUT_CH), lambda i: (0, 0)),
        ],
        out_specs=pl.BlockSpec((RB, OUT_CH), lambda i: (i, 0)),
        out_shape=jax.ShapeDtypeStruct((NPAD, OUT_CH), jnp.float32),
    )(s_cat, s_cat, deg, b.reshape(1, -1), w)


def _tc_sum2_body(p0_ref, p1_ref, o_ref):
    o_ref[...] = p0_ref[...] + p1_ref[...]


def _tc_sum2(parts):
    return pl.pallas_call(
        _tc_sum2_body,
        grid=(N_RB,),
        in_specs=[
            pl.BlockSpec((RB, OUT_CH), lambda i: (i, 0)),
            pl.BlockSpec((RB, OUT_CH), lambda i: (N_RB + i, 0)),
        ],
        out_specs=pl.BlockSpec((RB, OUT_CH), lambda i: (i, 0)),
        out_shape=jax.ShapeDtypeStruct((NPAD, OUT_CH), jnp.float32),
    )(parts, parts)


def _tc_last_body(p0_ref, p1_ref, deg_ref, b_ref, o_ref):
    dinv = lax.rsqrt(jnp.maximum(deg_ref[...], 1.0))
    s = p0_ref[...] + p1_ref[...]
    o_ref[...] = jnp.maximum(s * dinv + b_ref[...], 0.0)


def _tc_last(s_part, deg, b):
    return pl.pallas_call(
        _tc_last_body,
        grid=(N_RB,),
        in_specs=[
            pl.BlockSpec((RB, OUT_CH), lambda i: (i, 0)),
            pl.BlockSpec((RB, OUT_CH), lambda i: (N_RB + i, 0)),
            pl.BlockSpec((RB, 1), lambda i: (i, 0)),
            pl.BlockSpec((1, OUT_CH), lambda i: (0, 0)),
        ],
        out_specs=pl.BlockSpec((RB, OUT_CH), lambda i: (i, 0)),
        out_shape=jax.ShapeDtypeStruct((NPAD, OUT_CH), jnp.float32),
    )(s_part, s_part, deg, b.reshape(1, -1))


def kernel(x, edge_index, W1, b1, W2, b2, W3, b3, W4, b4, W5, b5):
    ei = edge_index.astype(jnp.int32)
    loops = jnp.arange(N, dtype=jnp.int32)
    src = jnp.concatenate([ei[0], loops])
    dst = jnp.concatenate([ei[1], loops])
    pad = EPAD - E_RAW
    src_p = jnp.concatenate([src, jnp.zeros((pad,), jnp.int32)])
    dst_p = jnp.concatenate([dst, jnp.full((pad,), N, jnp.int32)])
    src2 = jnp.concatenate([src_p, src_p + NPAD])

    nck = EPAD // CHUNK
    idt_ch = jnp.concatenate(
        [src2.reshape(2, nck, 1, CHUNK),
         jnp.broadcast_to(dst_p.reshape(1, nck, 1, CHUNK), (2, nck, 1, CHUNK))],
        axis=2).reshape(-1)
    idt_e = jnp.concatenate(
        [src_p.reshape(nck, 1, CHUNK), dst_p.reshape(nck, 1, CHUNK)],
        axis=1).reshape(-1)

    x_pad = jnp.zeros((NPAD, IN_CH), jnp.float32).at[:N].set(x)

    ones = jnp.ones((NPAD, 128), jnp.float32)
    deg_p = _sc_agg(128, True, True, ones, idt_e)
    deg = _tc_sum2(deg_p)[:, 0:1]

    g = _tc_first(x_pad, deg, W1)
    for wn, bn in ((W2, b1), (W3, b2), (W4, b3)):
        s_cat = _sc_agg(HID_CH // 2, False, False, g, idt_ch)
        g = _tc_mid(s_cat, deg, bn, wn)
    s_cat = _sc_agg(HID_CH // 2, False, False, g, idt_ch)
    g = _tc_mid_full(s_cat, deg, b4, W5)
    s_part = _sc_agg(OUT_CH, True, False, g, idt_e)
    out = _tc_last(s_part, deg, b5)
    return out[:N]


# speedup vs baseline: 1.1606x; 1.1606x over previous
import functools

import jax
import jax.numpy as jnp
from jax import lax
from jax.experimental import pallas as pl
from jax.experimental.pallas import tpu as pltpu
from jax.experimental.pallas import tpu_sc as plsc

N = 10000
IN_CH = 128
HID_CH = 256
OUT_CH = 128

NPAD = 10240
E_RAW = 320000 + N
CHUNK = 128
N_TILES = 16
EPAD = 335872
PER_TILE = EPAD // N_TILES
CHUNKS_PER_TILE = PER_TILE // CHUNK
ROWS_PER_TILE = NPAD // N_TILES
RB = 1024
N_RB = NPAD // RB


def _sc_agg_body(edge_split, skip_gather, table, idt, zeros, out, acc,
                 idt0, idt1, idt2, idt3, rows_a, rows_b, gsem, ssem,
                 lsem0, lsem1, lsem2, lsem3):
    c = lax.axis_index("c")
    s = lax.axis_index("s")
    idts = [idt0, idt1, idt2, idt3]
    lsems = [lsem0, lsem1, lsem2, lsem3]

    r0 = s * ROWS_PER_TILE
    pltpu.sync_copy(zeros.at[pl.ds(r0, ROWS_PER_TILE)],
                    acc.at[pl.ds(r0, ROWS_PER_TILE)])

    if edge_split:
        n_chunks = CHUNKS_PER_TILE // 2
        cchunk = c * (EPAD // 2 // CHUNK)
    else:
        n_chunks = CHUNKS_PER_TILE
        cchunk = c * (EPAD // CHUNK)

    if skip_gather:
        pltpu.async_copy(table.at[pl.ds(0, CHUNK)], rows_a, gsem).wait()

    def idt_start(j, q):
        off = (cchunk + s * n_chunks + j) * (2 * CHUNK)
        pltpu.async_copy(idt.at[pl.ds(off, 2 * CHUNK)], idts[q], lsems[q])

    def idt_wait(q):
        pltpu.make_async_copy(idt.at[pl.ds(0, 2 * CHUNK)], idts[q],
                              lsems[q]).wait()

    def scatter_wait():
        pltpu.make_async_copy(rows_a, acc.at[idt0.at[pl.ds(CHUNK, CHUNK)]],
                              ssem).wait()

    idt_start(0, 0)
    idt_start(1, 1)
    idt_start(2, 2)
    plsc.subcore_barrier()

    def block(j, q, buf, first=False):
        idt_wait(q)
        if not skip_gather:
            pltpu.async_copy(table.at[idts[q].at[pl.ds(0, CHUNK)]], buf,
                             gsem).wait()
        else:
            buf = rows_a
        if not first:
            scatter_wait()
        idt_start(jnp.minimum(j + 3, n_chunks - 1), (q + 3) % 4)
        pltpu.async_copy(buf, acc.at[idts[q].at[pl.ds(CHUNK, CHUNK)]],
                         ssem, add=True)

    block(0, 0, rows_a, first=True)
    block(1, 1, rows_b)
    block(2, 2, rows_a)
    block(3, 3, rows_b)

    def group(p, carry):
        base = 4 * p
        block(base, 0, rows_a)
        block(base + 1, 1, rows_b)
        block(base + 2, 2, rows_a)
        block(base + 3, 3, rows_b)
        return carry

    lax.fori_loop(1, n_chunks // 4, group, 0)
    for j in range(n_chunks - (n_chunks % 4), n_chunks):
        block(j, j % 4, rows_a if j % 2 == 0 else rows_b)
    scatter_wait()
    for q in (n_chunks % 4, (n_chunks + 1) % 4, (n_chunks + 2) % 4):
        idt_wait(q)
    plsc.subcore_barrier()

    pltpu.sync_copy(acc.at[pl.ds(r0, ROWS_PER_TILE)],
                    out.at[pl.ds(c * NPAD + r0, ROWS_PER_TILE)])


@functools.partial(jax.jit, static_argnums=(0, 1, 2))
def _sc_agg(h, edge_split, skip_gather, table, idt):
    zeros = jnp.zeros((NPAD, h), jnp.float32)
    mesh = plsc.VectorSubcoreMesh(core_axis_name="c", subcore_axis_name="s")
    return pl.kernel(
        functools.partial(_sc_agg_body, edge_split, skip_gather),
        out_type=jax.ShapeDtypeStruct((2 * NPAD, h), jnp.float32),
        mesh=mesh,
        scratch_types=[
            pltpu.VMEM_SHARED((NPAD, h), jnp.float32),
            pltpu.VMEM((2 * CHUNK,), jnp.int32),
            pltpu.VMEM((2 * CHUNK,), jnp.int32),
            pltpu.VMEM((2 * CHUNK,), jnp.int32),
            pltpu.VMEM((2 * CHUNK,), jnp.int32),
            pltpu.VMEM((CHUNK, h), jnp.float32),
            pltpu.VMEM((CHUNK, h), jnp.float32),
            pltpu.SemaphoreType.DMA,
            pltpu.SemaphoreType.DMA,
            pltpu.SemaphoreType.DMA,
            pltpu.SemaphoreType.DMA,
            pltpu.SemaphoreType.DMA,
            pltpu.SemaphoreType.DMA,
        ],
    )(table, idt, zeros)


def _tc_first_body(x_ref, deg_ref, w_ref, o_ref):
    dinv = lax.rsqrt(jnp.maximum(deg_ref[...], 1.0))
    t = x_ref[...] * dinv
    o_ref[...] = jnp.dot(t, w_ref[...], preferred_element_type=jnp.float32)


def _tc_first(x_pad, deg, w1):
    return pl.pallas_call(
        _tc_first_body,
        grid=(2, N_RB),
        in_specs=[
            pl.BlockSpec((RB, IN_CH), lambda c, i: (i, 0)),
            pl.BlockSpec((RB, 1), lambda c, i: (i, 0)),
            pl.BlockSpec((IN_CH, HID_CH // 2), lambda c, i: (0, c)),
        ],
        out_specs=pl.BlockSpec((RB, HID_CH // 2), lambda c, i: (c * N_RB + i, 0)),
        out_shape=jax.ShapeDtypeStruct((2 * NPAD, HID_CH // 2), jnp.float32),
    )(x_pad, deg, w1)


def _tc_mid_body(lo_ref, hi_ref, deg_ref, b_ref, w_ref, o_ref):
    dinv = lax.rsqrt(jnp.maximum(deg_ref[...], 1.0))
    s = jnp.concatenate([lo_ref[...], hi_ref[...]], axis=1)
    u = jnp.maximum(s * dinv + b_ref[...], 0.0)
    t = u * dinv
    o_ref[...] = jnp.dot(t, w_ref[...], preferred_element_type=jnp.float32)


def _tc_mid(s_cat, deg, b, w):
    h = HID_CH // 2
    return pl.pallas_call(
        _tc_mid_body,
        grid=(2, N_RB),
        in_specs=[
            pl.BlockSpec((RB, h), lambda c, i: (i, 0)),
            pl.BlockSpec((RB, h), lambda c, i: (N_RB + i, 0)),
            pl.BlockSpec((RB, 1), lambda c, i: (i, 0)),
            pl.BlockSpec((1, HID_CH), lambda c, i: (0, 0)),
            pl.BlockSpec((HID_CH, h), lambda c, i: (0, c)),
        ],
        out_specs=pl.BlockSpec((RB, h), lambda c, i: (c * N_RB + i, 0)),
        out_shape=jax.ShapeDtypeStruct((2 * NPAD, h), jnp.float32),
    )(s_cat, s_cat, deg, b.reshape(1, -1), w)


def _tc_mid_full(s_cat, deg, b, w):
    h = HID_CH // 2
    return pl.pallas_call(
        _tc_mid_body,
        grid=(N_RB,),
        in_specs=[
            pl.BlockSpec((RB, h), lambda i: (i, 0)),
            pl.BlockSpec((RB, h), lambda i: (N_RB + i, 0)),
            pl.BlockSpec((RB, 1), lambda i: (i, 0)),
            pl.BlockSpec((1, HID_CH), lambda i: (0, 0)),
            pl.BlockSpec((HID_CH, OUT_CH), lambda i: (0, 0)),
        ],
        out_specs=pl.BlockSpec((RB, OUT_CH), lambda i: (i, 0)),
        out_shape=jax.ShapeDtypeStruct((NPAD, OUT_CH), jnp.float32),
    )(s_cat, s_cat, deg, b.reshape(1, -1), w)


def _tc_sum2_body(p0_ref, p1_ref, o_ref):
    o_ref[...] = p0_ref[...] + p1_ref[...]


def _tc_sum2(parts):
    return pl.pallas_call(
        _tc_sum2_body,
        grid=(N_RB,),
        in_specs=[
            pl.BlockSpec((RB, OUT_CH), lambda i: (i, 0)),
            pl.BlockSpec((RB, OUT_CH), lambda i: (N_RB + i, 0)),
        ],
        out_specs=pl.BlockSpec((RB, OUT_CH), lambda i: (i, 0)),
        out_shape=jax.ShapeDtypeStruct((NPAD, OUT_CH), jnp.float32),
    )(parts, parts)


def _tc_last_body(p0_ref, p1_ref, deg_ref, b_ref, o_ref):
    dinv = lax.rsqrt(jnp.maximum(deg_ref[...], 1.0))
    s = p0_ref[...] + p1_ref[...]
    o_ref[...] = jnp.maximum(s * dinv + b_ref[...], 0.0)


def _tc_last(s_part, deg, b):
    return pl.pallas_call(
        _tc_last_body,
        grid=(N_RB,),
        in_specs=[
            pl.BlockSpec((RB, OUT_CH), lambda i: (i, 0)),
            pl.BlockSpec((RB, OUT_CH), lambda i: (N_RB + i, 0)),
            pl.BlockSpec((RB, 1), lambda i: (i, 0)),
            pl.BlockSpec((1, OUT_CH), lambda i: (0, 0)),
        ],
        out_specs=pl.BlockSpec((RB, OUT_CH), lambda i: (i, 0)),
        out_shape=jax.ShapeDtypeStruct((NPAD, OUT_CH), jnp.float32),
    )(s_part, s_part, deg, b.reshape(1, -1))


def kernel(x, edge_index, W1, b1, W2, b2, W3, b3, W4, b4, W5, b5):
    ei = edge_index.astype(jnp.int32)
    loops = jnp.arange(N, dtype=jnp.int32)
    src = jnp.concatenate([ei[0], loops])
    dst = jnp.concatenate([ei[1], loops])
    pad = EPAD - E_RAW
    src_p = jnp.concatenate([src, jnp.zeros((pad,), jnp.int32)])
    dst_p = jnp.concatenate([dst, jnp.full((pad,), N, jnp.int32)])
    src2 = jnp.concatenate([src_p, src_p + NPAD])

    nck = EPAD // CHUNK
    idt_ch = jnp.concatenate(
        [src2.reshape(2, nck, 1, CHUNK),
         jnp.broadcast_to(dst_p.reshape(1, nck, 1, CHUNK), (2, nck, 1, CHUNK))],
        axis=2).reshape(-1)
    idt_e = jnp.concatenate(
        [src_p.reshape(nck, 1, CHUNK), dst_p.reshape(nck, 1, CHUNK)],
        axis=1).reshape(-1)

    x_pad = jnp.zeros((NPAD, IN_CH), jnp.float32).at[:N].set(x)

    ones = jnp.ones((NPAD, 128), jnp.float32)
    deg_p = _sc_agg(128, True, True, ones, idt_e)
    deg = _tc_sum2(deg_p)[:, 0:1]

    g = _tc_first(x_pad, deg, W1)
    for wn, bn in ((W2, b1), (W3, b2), (W4, b3)):
        s_cat = _sc_agg(HID_CH // 2, False, False, g, idt_ch)
        g = _tc_mid(s_cat, deg, bn, wn)
    s_cat = _sc_agg(HID_CH // 2, False, False, g, idt_ch)
    g = _tc_mid_full(s_cat, deg, b4, W5)
    s_part = _sc_agg(OUT_CH, True, False, g, idt_e)
    out = _tc_last(s_part, deg, b5)
    return out[:N]


# speedup vs baseline: 1.1754x; 1.0127x over previous
import functools

import jax
import jax.numpy as jnp
from jax import lax
from jax.experimental import pallas as pl
from jax.experimental.pallas import tpu as pltpu
from jax.experimental.pallas import tpu_sc as plsc

N = 10000
IN_CH = 128
HID_CH = 256
OUT_CH = 128

NPAD = 10240
E_RAW = 320000 + N
CHUNK = 128
N_TILES = 16
EPAD = 335872
PER_TILE = EPAD // N_TILES
CHUNKS_PER_TILE = PER_TILE // CHUNK
ROWS_PER_TILE = NPAD // N_TILES
RB = 1024
N_RB = NPAD // RB


def _sc_agg_body(edge_split, skip_gather, table, idt, zeros, out, acc,
                 idt0, idt1, idt2, idt3, rows_a, rows_b, gsem, ssem,
                 lsem0, lsem1, lsem2, lsem3):
    c = lax.axis_index("c")
    s = lax.axis_index("s")
    idts = [idt0, idt1, idt2, idt3]
    lsems = [lsem0, lsem1, lsem2, lsem3]

    r0 = s * ROWS_PER_TILE
    pltpu.sync_copy(zeros.at[pl.ds(r0, ROWS_PER_TILE)],
                    acc.at[pl.ds(r0, ROWS_PER_TILE)])

    if edge_split:
        n_chunks = CHUNKS_PER_TILE // 2
    else:
        n_chunks = CHUNKS_PER_TILE
        cchunk = c * (EPAD // CHUNK)

    if skip_gather:
        pltpu.async_copy(table.at[pl.ds(0, CHUNK)], rows_a, gsem).wait()

    def idt_start(j, q):
        if edge_split:
            off = ((s * n_chunks + j) * 2 + c) * (2 * CHUNK)
        else:
            off = (cchunk + s * n_chunks + j) * (2 * CHUNK)
        pltpu.async_copy(idt.at[pl.ds(off, 2 * CHUNK)], idts[q], lsems[q])

    def idt_wait(q):
        pltpu.make_async_copy(idt.at[pl.ds(0, 2 * CHUNK)], idts[q],
                              lsems[q]).wait()

    def scatter_wait():
        pltpu.make_async_copy(rows_a, acc.at[idt0.at[pl.ds(CHUNK, CHUNK)]],
                              ssem).wait()

    idt_start(0, 0)
    idt_start(1, 1)
    idt_start(2, 2)
    plsc.subcore_barrier()

    def block(j, q, buf, first=False):
        idt_wait(q)
        if not skip_gather:
            pltpu.async_copy(table.at[idts[q].at[pl.ds(0, CHUNK)]], buf,
                             gsem).wait()
        else:
            buf = rows_a
        if not first:
            scatter_wait()
        idt_start(jnp.minimum(j + 3, n_chunks - 1), (q + 3) % 4)
        pltpu.async_copy(buf, acc.at[idts[q].at[pl.ds(CHUNK, CHUNK)]],
                         ssem, add=True)

    block(0, 0, rows_a, first=True)
    block(1, 1, rows_b)
    block(2, 2, rows_a)
    block(3, 3, rows_b)

    def group(p, carry):
        base = 4 * p
        block(base, 0, rows_a)
        block(base + 1, 1, rows_b)
        block(base + 2, 2, rows_a)
        block(base + 3, 3, rows_b)
        return carry

    lax.fori_loop(1, n_chunks // 4, group, 0)
    for j in range(n_chunks - (n_chunks % 4), n_chunks):
        block(j, j % 4, rows_a if j % 2 == 0 else rows_b)
    scatter_wait()
    for q in (n_chunks % 4, (n_chunks + 1) % 4, (n_chunks + 2) % 4):
        idt_wait(q)
    plsc.subcore_barrier()

    pltpu.sync_copy(acc.at[pl.ds(r0, ROWS_PER_TILE)],
                    out.at[pl.ds(c * NPAD + r0, ROWS_PER_TILE)])


@functools.partial(jax.jit, static_argnums=(0, 1, 2))
def _sc_agg(h, edge_split, skip_gather, table, idt):
    zeros = jnp.zeros((NPAD, h), jnp.float32)
    mesh = plsc.VectorSubcoreMesh(core_axis_name="c", subcore_axis_name="s")
    return pl.kernel(
        functools.partial(_sc_agg_body, edge_split, skip_gather),
        out_type=jax.ShapeDtypeStruct((2 * NPAD, h), jnp.float32),
        mesh=mesh,
        scratch_types=[
            pltpu.VMEM_SHARED((NPAD, h), jnp.float32),
            pltpu.VMEM((2 * CHUNK,), jnp.int32),
            pltpu.VMEM((2 * CHUNK,), jnp.int32),
            pltpu.VMEM((2 * CHUNK,), jnp.int32),
            pltpu.VMEM((2 * CHUNK,), jnp.int32),
            pltpu.VMEM((CHUNK, h), jnp.float32),
            pltpu.VMEM((CHUNK, h), jnp.float32),
            pltpu.SemaphoreType.DMA,
            pltpu.SemaphoreType.DMA,
            pltpu.SemaphoreType.DMA,
            pltpu.SemaphoreType.DMA,
            pltpu.SemaphoreType.DMA,
            pltpu.SemaphoreType.DMA,
        ],
    )(table, idt, zeros)


def _tc_first_body(x_ref, deg_ref, w_ref, o_ref):
    dinv = lax.rsqrt(jnp.maximum(deg_ref[...], 1.0))
    t = x_ref[...] * dinv
    o_ref[...] = jnp.dot(t, w_ref[...], preferred_element_type=jnp.float32)


def _tc_first(x_pad, deg, w1):
    return pl.pallas_call(
        _tc_first_body,
        grid=(2, N_RB),
        in_specs=[
            pl.BlockSpec((RB, IN_CH), lambda c, i: (i, 0)),
            pl.BlockSpec((RB, 1), lambda c, i: (i, 0)),
            pl.BlockSpec((IN_CH, HID_CH // 2), lambda c, i: (0, c)),
        ],
        out_specs=pl.BlockSpec((RB, HID_CH // 2), lambda c, i: (c * N_RB + i, 0)),
        out_shape=jax.ShapeDtypeStruct((2 * NPAD, HID_CH // 2), jnp.float32),
    )(x_pad, deg, w1)


def _tc_mid_body(lo_ref, hi_ref, deg_ref, b_ref, w_ref, o_ref):
    dinv = lax.rsqrt(jnp.maximum(deg_ref[...], 1.0))
    s = jnp.concatenate([lo_ref[...], hi_ref[...]], axis=1)
    u = jnp.maximum(s * dinv + b_ref[...], 0.0)
    t = u * dinv
    o_ref[...] = jnp.dot(t, w_ref[...], preferred_element_type=jnp.float32)


def _tc_mid(s_cat, deg, b, w):
    h = HID_CH // 2
    return pl.pallas_call(
        _tc_mid_body,
        grid=(2, N_RB),
        in_specs=[
            pl.BlockSpec((RB, h), lambda c, i: (i, 0)),
            pl.BlockSpec((RB, h), lambda c, i: (N_RB + i, 0)),
            pl.BlockSpec((RB, 1), lambda c, i: (i, 0)),
            pl.BlockSpec((1, HID_CH), lambda c, i: (0, 0)),
            pl.BlockSpec((HID_CH, h), lambda c, i: (0, c)),
        ],
        out_specs=pl.BlockSpec((RB, h), lambda c, i: (c * N_RB + i, 0)),
        out_shape=jax.ShapeDtypeStruct((2 * NPAD, h), jnp.float32),
    )(s_cat, s_cat, deg, b.reshape(1, -1), w)


def _tc_mid_full(s_cat, deg, b, w):
    h = HID_CH // 2
    return pl.pallas_call(
        _tc_mid_body,
        grid=(N_RB,),
        in_specs=[
            pl.BlockSpec((RB, h), lambda i: (i, 0)),
            pl.BlockSpec((RB, h), lambda i: (N_RB + i, 0)),
            pl.BlockSpec((RB, 1), lambda i: (i, 0)),
            pl.BlockSpec((1, HID_CH), lambda i: (0, 0)),
            pl.BlockSpec((HID_CH, OUT_CH), lambda i: (0, 0)),
        ],
        out_specs=pl.BlockSpec((RB, OUT_CH), lambda i: (i, 0)),
        out_shape=jax.ShapeDtypeStruct((NPAD, OUT_CH), jnp.float32),
    )(s_cat, s_cat, deg, b.reshape(1, -1), w)


def _tc_sum2_body(p0_ref, p1_ref, o_ref):
    o_ref[...] = p0_ref[...] + p1_ref[...]


def _tc_sum2(parts):
    return pl.pallas_call(
        _tc_sum2_body,
        grid=(N_RB,),
        in_specs=[
            pl.BlockSpec((RB, OUT_CH), lambda i: (i, 0)),
            pl.BlockSpec((RB, OUT_CH), lambda i: (N_RB + i, 0)),
        ],
        out_specs=pl.BlockSpec((RB, OUT_CH), lambda i: (i, 0)),
        out_shape=jax.ShapeDtypeStruct((NPAD, OUT_CH), jnp.float32),
    )(parts, parts)


def _tc_last_body(p0_ref, p1_ref, deg_ref, b_ref, o_ref):
    dinv = lax.rsqrt(jnp.maximum(deg_ref[...], 1.0))
    s = p0_ref[...] + p1_ref[...]
    o_ref[...] = jnp.maximum(s * dinv + b_ref[...], 0.0)


def _tc_last(s_part, deg, b):
    return pl.pallas_call(
        _tc_last_body,
        grid=(N_RB,),
        in_specs=[
            pl.BlockSpec((RB, OUT_CH), lambda i: (i, 0)),
            pl.BlockSpec((RB, OUT_CH), lambda i: (N_RB + i, 0)),
            pl.BlockSpec((RB, 1), lambda i: (i, 0)),
            pl.BlockSpec((1, OUT_CH), lambda i: (0, 0)),
        ],
        out_specs=pl.BlockSpec((RB, OUT_CH), lambda i: (i, 0)),
        out_shape=jax.ShapeDtypeStruct((NPAD, OUT_CH), jnp.float32),
    )(s_part, s_part, deg, b.reshape(1, -1))


def kernel(x, edge_index, W1, b1, W2, b2, W3, b3, W4, b4, W5, b5):
    ei = edge_index.astype(jnp.int32)
    loops = jnp.arange(N, dtype=jnp.int32)
    src = jnp.concatenate([ei[0], loops])
    dst = jnp.concatenate([ei[1], loops])
    pad = EPAD - E_RAW
    src_p = jnp.concatenate([src, jnp.zeros((pad,), jnp.int32)])
    dst_p = jnp.concatenate([dst, jnp.full((pad,), N, jnp.int32)])
    src2 = jnp.concatenate([src_p, src_p + NPAD])

    nck = EPAD // CHUNK
    idt_ch = jnp.concatenate(
        [src2.reshape(2, nck, 1, CHUNK),
         jnp.broadcast_to(dst_p.reshape(1, nck, 1, CHUNK), (2, nck, 1, CHUNK))],
        axis=2).reshape(-1)
    idt_e = jnp.concatenate(
        [src_p.reshape(nck, 1, CHUNK), dst_p.reshape(nck, 1, CHUNK)],
        axis=1).reshape(-1)

    x_pad = jnp.zeros((NPAD, IN_CH), jnp.float32).at[:N].set(x)

    ones = jnp.ones((NPAD, 128), jnp.float32)
    deg_p = _sc_agg(128, True, True, ones, idt_e)
    deg = _tc_sum2(deg_p)[:, 0:1]

    g = _tc_first(x_pad, deg, W1)
    for wn, bn in ((W2, b1), (W3, b2), (W4, b3)):
        s_cat = _sc_agg(HID_CH // 2, False, False, g, idt_ch)
        g = _tc_mid(s_cat, deg, bn, wn)
    s_cat = _sc_agg(HID_CH // 2, False, False, g, idt_ch)
    g = _tc_mid_full(s_cat, deg, b4, W5)
    s_part = _sc_agg(OUT_CH, True, False, g, idt_e)
    out = _tc_last(s_part, deg, b5)
    return out[:N]


# speedup vs baseline: 1.1788x; 1.0029x over previous
import functools

import jax
import jax.numpy as jnp
from jax import lax
from jax.experimental import pallas as pl
from jax.experimental.pallas import tpu as pltpu
from jax.experimental.pallas import tpu_sc as plsc

N = 10000
IN_CH = 128
HID_CH = 256
OUT_CH = 128

NPAD = 10240
E_RAW = 320000 + N
CHUNK = 128
N_TILES = 16
EPAD = 335872
PER_TILE = EPAD // N_TILES
CHUNKS_PER_TILE = PER_TILE // CHUNK
ROWS_PER_TILE = NPAD // N_TILES
RB = 1024
N_RB = NPAD // RB


def _sc_agg_body(edge_split, skip_gather, table, idt, zeros, out, acc,
                 idt0, idt1, idt2, idt3, rows_a, rows_b, gsem, ssem_a, ssem_b,
                 lsem0, lsem1, lsem2, lsem3):
    c = lax.axis_index("c")
    s = lax.axis_index("s")
    idts = [idt0, idt1, idt2, idt3]
    lsems = [lsem0, lsem1, lsem2, lsem3]

    r0 = s * ROWS_PER_TILE
    pltpu.sync_copy(zeros.at[pl.ds(r0, ROWS_PER_TILE)],
                    acc.at[pl.ds(r0, ROWS_PER_TILE)])

    if edge_split:
        n_chunks = CHUNKS_PER_TILE // 2
    else:
        n_chunks = CHUNKS_PER_TILE
        cchunk = c * (EPAD // CHUNK)

    if skip_gather:
        pltpu.async_copy(table.at[pl.ds(0, CHUNK)], rows_a, gsem).wait()

    def idt_start(j, q):
        if edge_split:
            off = ((s * n_chunks + j) * 2 + c) * (2 * CHUNK)
        else:
            off = (cchunk + s * n_chunks + j) * (2 * CHUNK)
        pltpu.async_copy(idt.at[pl.ds(off, 2 * CHUNK)], idts[q], lsems[q])

    def idt_wait(q):
        pltpu.make_async_copy(idt.at[pl.ds(0, 2 * CHUNK)], idts[q],
                              lsems[q]).wait()

    def scatter_wait(sem):
        pltpu.make_async_copy(rows_a, acc.at[idt0.at[pl.ds(CHUNK, CHUNK)]],
                              sem).wait()

    idt_start(0, 0)
    idt_start(1, 1)
    plsc.subcore_barrier()

    def block(j, q, buf, sem, first=False):
        if not first:
            scatter_wait(sem)
        idt_wait(q)
        if not skip_gather:
            pltpu.async_copy(table.at[idts[q].at[pl.ds(0, CHUNK)]], buf,
                             gsem).wait()
        else:
            buf = rows_a
        idt_start(jnp.minimum(j + 2, n_chunks - 1), (q + 2) % 4)
        pltpu.async_copy(buf, acc.at[idts[q].at[pl.ds(CHUNK, CHUNK)]],
                         sem, add=True)

    block(0, 0, rows_a, ssem_a, first=True)
    block(1, 1, rows_b, ssem_b, first=True)
    block(2, 2, rows_a, ssem_a)
    block(3, 3, rows_b, ssem_b)

    def group(p, carry):
        base = 4 * p
        block(base, 0, rows_a, ssem_a)
        block(base + 1, 1, rows_b, ssem_b)
        block(base + 2, 2, rows_a, ssem_a)
        block(base + 3, 3, rows_b, ssem_b)
        return carry

    lax.fori_loop(1, n_chunks // 4, group, 0)
    for j in range(n_chunks - (n_chunks % 4), n_chunks):
        block(j, j % 4, rows_a if j % 2 == 0 else rows_b,
              ssem_a if j % 2 == 0 else ssem_b)
    scatter_wait(ssem_a)
    scatter_wait(ssem_b)
    for q in (n_chunks % 4, (n_chunks + 1) % 4):
        idt_wait(q)
    plsc.subcore_barrier()

    pltpu.sync_copy(acc.at[pl.ds(r0, ROWS_PER_TILE)],
                    out.at[pl.ds(c * NPAD + r0, ROWS_PER_TILE)])


@functools.partial(jax.jit, static_argnums=(0, 1, 2))
def _sc_agg(h, edge_split, skip_gather, table, idt):
    zeros = jnp.zeros((NPAD, h), jnp.float32)
    mesh = plsc.VectorSubcoreMesh(core_axis_name="c", subcore_axis_name="s")
    return pl.kernel(
        functools.partial(_sc_agg_body, edge_split, skip_gather),
        out_type=jax.ShapeDtypeStruct((2 * NPAD, h), jnp.float32),
        mesh=mesh,
        scratch_types=[
            pltpu.VMEM_SHARED((NPAD, h), jnp.float32),
            pltpu.VMEM((2 * CHUNK,), jnp.int32),
            pltpu.VMEM((2 * CHUNK,), jnp.int32),
            pltpu.VMEM((2 * CHUNK,), jnp.int32),
            pltpu.VMEM((2 * CHUNK,), jnp.int32),
            pltpu.VMEM((CHUNK, h), jnp.float32),
            pltpu.VMEM((CHUNK, h), jnp.float32),
            pltpu.SemaphoreType.DMA,
            pltpu.SemaphoreType.DMA,
            pltpu.SemaphoreType.DMA,
            pltpu.SemaphoreType.DMA,
            pltpu.SemaphoreType.DMA,
            pltpu.SemaphoreType.DMA,
            pltpu.SemaphoreType.DMA,
        ],
    )(table, idt, zeros)


def _tc_first_body(x_ref, deg_ref, w_ref, o_ref):
    dinv = lax.rsqrt(jnp.maximum(deg_ref[...], 1.0))
    t = x_ref[...] * dinv
    o_ref[...] = jnp.dot(t, w_ref[...], preferred_element_type=jnp.float32)


def _tc_first(x_pad, deg, w1):
    return pl.pallas_call(
        _tc_first_body,
        grid=(2, N_RB),
        in_specs=[
            pl.BlockSpec((RB, IN_CH), lambda c, i: (i, 0)),
            pl.BlockSpec((RB, 1), lambda c, i: (i, 0)),
            pl.BlockSpec((IN_CH, HID_CH // 2), lambda c, i: (0, c)),
        ],
        out_specs=pl.BlockSpec((RB, HID_CH // 2), lambda c, i: (c * N_RB + i, 0)),
        out_shape=jax.ShapeDtypeStruct((2 * NPAD, HID_CH // 2), jnp.float32),
    )(x_pad, deg, w1)


def _tc_mid_body(lo_ref, hi_ref, deg_ref, b_ref, w_ref, o_ref):
    dinv = lax.rsqrt(jnp.maximum(deg_ref[...], 1.0))
    s = jnp.concatenate([lo_ref[...], hi_ref[...]], axis=1)
    u = jnp.maximum(s * dinv + b_ref[...], 0.0)
    t = u * dinv
    o_ref[...] = jnp.dot(t, w_ref[...], preferred_element_type=jnp.float32)


def _tc_mid(s_cat, deg, b, w):
    h = HID_CH // 2
    return pl.pallas_call(
        _tc_mid_body,
        grid=(2, N_RB),
        in_specs=[
            pl.BlockSpec((RB, h), lambda c, i: (i, 0)),
            pl.BlockSpec((RB, h), lambda c, i: (N_RB + i, 0)),
            pl.BlockSpec((RB, 1), lambda c, i: (i, 0)),
            pl.BlockSpec((1, HID_CH), lambda c, i: (0, 0)),
            pl.BlockSpec((HID_CH, h), lambda c, i: (0, c)),
        ],
        out_specs=pl.BlockSpec((RB, h), lambda c, i: (c * N_RB + i, 0)),
        out_shape=jax.ShapeDtypeStruct((2 * NPAD, h), jnp.float32),
    )(s_cat, s_cat, deg, b.reshape(1, -1), w)


def _tc_mid_full(s_cat, deg, b, w):
    h = HID_CH // 2
    return pl.pallas_call(
        _tc_mid_body,
        grid=(N_RB,),
        in_specs=[
            pl.BlockSpec((RB, h), lambda i: (i, 0)),
            pl.BlockSpec((RB, h), lambda i: (N_RB + i, 0)),
            pl.BlockSpec((RB, 1), lambda i: (i, 0)),
            pl.BlockSpec((1, HID_CH), lambda i: (0, 0)),
            pl.BlockSpec((HID_CH, OUT_CH), lambda i: (0, 0)),
        ],
        out_specs=pl.BlockSpec((RB, OUT_CH), lambda i: (i, 0)),
        out_shape=jax.ShapeDtypeStruct((NPAD, OUT_CH), jnp.float32),
    )(s_cat, s_cat, deg, b.reshape(1, -1), w)


def _tc_sum2_body(p0_ref, p1_ref, o_ref):
    o_ref[...] = p0_ref[...] + p1_ref[...]


def _tc_sum2(parts):
    return pl.pallas_call(
        _tc_sum2_body,
        grid=(N_RB,),
        in_specs=[
            pl.BlockSpec((RB, OUT_CH), lambda i: (i, 0)),
            pl.BlockSpec((RB, OUT_CH), lambda i: (N_RB + i, 0)),
        ],
        out_specs=pl.BlockSpec((RB, OUT_CH), lambda i: (i, 0)),
        out_shape=jax.ShapeDtypeStruct((NPAD, OUT_CH), jnp.float32),
    )(parts, parts)


def _tc_last_body(p0_ref, p1_ref, deg_ref, b_ref, o_ref):
    dinv = lax.rsqrt(jnp.maximum(deg_ref[...], 1.0))
    s = p0_ref[...] + p1_ref[...]
    o_ref[...] = jnp.maximum(s * dinv + b_ref[...], 0.0)


def _tc_last(s_part, deg, b):
    return pl.pallas_call(
        _tc_last_body,
        grid=(N_RB,),
        in_specs=[
            pl.BlockSpec((RB, OUT_CH), lambda i: (i, 0)),
            pl.BlockSpec((RB, OUT_CH), lambda i: (N_RB + i, 0)),
            pl.BlockSpec((RB, 1), lambda i: (i, 0)),
            pl.BlockSpec((1, OUT_CH), lambda i: (0, 0)),
        ],
        out_specs=pl.BlockSpec((RB, OUT_CH), lambda i: (i, 0)),
        out_shape=jax.ShapeDtypeStruct((NPAD, OUT_CH), jnp.float32),
    )(s_part, s_part, deg, b.reshape(1, -1))


def kernel(x, edge_index, W1, b1, W2, b2, W3, b3, W4, b4, W5, b5):
    ei = edge_index.astype(jnp.int32)
    loops = jnp.arange(N, dtype=jnp.int32)
    src = jnp.concatenate([ei[0], loops])
    dst = jnp.concatenate([ei[1], loops])
    pad = EPAD - E_RAW
    src_p = jnp.concatenate([src, jnp.zeros((pad,), jnp.int32)])
    dst_p = jnp.concatenate([dst, jnp.full((pad,), N, jnp.int32)])
    src2 = jnp.concatenate([src_p, src_p + NPAD])

    nck = EPAD // CHUNK
    idt_ch = jnp.concatenate(
        [src2.reshape(2, nck, 1, CHUNK),
         jnp.broadcast_to(dst_p.reshape(1, nck, 1, CHUNK), (2, nck, 1, CHUNK))],
        axis=2).reshape(-1)
    idt_e = jnp.concatenate(
        [src_p.reshape(nck, 1, CHUNK), dst_p.reshape(nck, 1, CHUNK)],
        axis=1).reshape(-1)

    x_pad = jnp.zeros((NPAD, IN_CH), jnp.float32).at[:N].set(x)

    ones = jnp.ones((NPAD, 128), jnp.float32)
    deg_p = _sc_agg(128, True, True, ones, idt_e)
    deg = _tc_sum2(deg_p)[:, 0:1]

    g = _tc_first(x_pad, deg, W1)
    for wn, bn in ((W2, b1), (W3, b2), (W4, b3)):
        s_cat = _sc_agg(HID_CH // 2, False, False, g, idt_ch)
        g = _tc_mid(s_cat, deg, bn, wn)
    s_cat = _sc_agg(HID_CH // 2, False, False, g, idt_ch)
    g = _tc_mid_full(s_cat, deg, b4, W5)
    s_part = _sc_agg(OUT_CH, True, False, g, idt_e)
    out = _tc_last(s_part, deg, b5)
    return out[:N]
